# baseline (speedup 1.0000x reference)
import jax
import jax.numpy as jnp
from jax.experimental import pallas as pl
from jax.experimental.pallas import tpu as pltpu

EPS = 1e-10
_TBLK = 256


def _body(pred_ref, tar_ref, mask_ref, out_ref, s_acc, c_acc):
    b = pl.program_id(0)
    t = pl.program_id(1)
    step = b * pl.num_programs(1) + t
    F = tar_ref.shape[-1]

    @pl.when(step == 0)
    def _():
        s_acc[...] = jnp.zeros_like(s_acc)
        c_acc[...] = jnp.zeros_like(c_acc)

    m = mask_ref[...].reshape(1, _TBLK, 1).astype(jnp.float32)
    t_log = jnp.log(tar_ref[...] + EPS)
    d = jnp.abs(pred_ref[...] - t_log) * m
    s_acc[...] += jnp.sum(d.reshape(_TBLK // 8, 8, F), axis=0)
    c_acc[...] += jnp.sum(m.reshape(_TBLK // 8, 8, 1), axis=0)

    @pl.when(step == pl.num_programs(0) * pl.num_programs(1) - 1)
    def _():
        out_ref[...] = (jnp.sum(s_acc[...]) / (jnp.sum(c_acc[...]) * F)).reshape(1, 1)


def kernel(log_predicted, linear_tar, stft_length_masks):
    B, T, F = log_predicted.shape
    mask3 = stft_length_masks.reshape(B, 1, T)

    out = pl.pallas_call(
        _body,
        grid=(B, T // _TBLK),
        in_specs=[
            pl.BlockSpec((1, _TBLK, F), lambda b, t: (b, t, 0)),
            pl.BlockSpec((1, _TBLK, F), lambda b, t: (b, t, 0)),
            pl.BlockSpec((1, 1, _TBLK), lambda b, t: (b, 0, t)),
        ],
        out_specs=pl.BlockSpec((1, 1), lambda b, t: (0, 0)),
        out_shape=jax.ShapeDtypeStruct((1, 1), jnp.float32),
        scratch_shapes=[
            pltpu.VMEM((8, F), jnp.float32),
            pltpu.VMEM((8, 1), jnp.float32),
        ],
    )(log_predicted, linear_tar, mask3)
    return out[0, 0]
